# baseline (speedup 1.0000x reference)
import functools

import jax
import jax.numpy as jnp
from jax import lax
from jax.experimental import pallas as pl
from jax.experimental.pallas import tpu as pltpu
from jax.experimental.pallas import tpu_sc as plsc

VOCAB = 1_000_000
D = 64
B = 4096 * 50
IDX_MINOR = 128
N_ROWS = B // IDX_MINOR
NW = 32
ROWS_PER_W = N_ROWS // NW


def _gather_body(idx_hbm, table_hbm, out_hbm, idx_v, rows_v, sem):
    wid = lax.axis_index("s") * 2 + lax.axis_index("c")
    row0 = wid * ROWS_PER_W
    pltpu.sync_copy(idx_hbm.at[wid], idx_v)

    def step(j, carry):
        pltpu.async_copy(table_hbm.at[idx_v.at[j]], rows_v, sem).wait()
        pltpu.sync_copy(rows_v,
                        out_hbm.at[pl.ds((row0 + j) * IDX_MINOR, IDX_MINOR)])
        return carry

    lax.fori_loop(0, ROWS_PER_W, step, 0)


@jax.jit
def _embed(idx3, table):
    run = functools.partial(
        pl.kernel,
        mesh=plsc.VectorSubcoreMesh(core_axis_name="c", subcore_axis_name="s"),
        out_type=jax.ShapeDtypeStruct((B, D), jnp.float32),
        scratch_types=[
            pltpu.VMEM((ROWS_PER_W, IDX_MINOR), jnp.int32),
            pltpu.VMEM((IDX_MINOR, D), jnp.float32),
            pltpu.SemaphoreType.DMA,
        ],
        compiler_params=pltpu.CompilerParams(use_tc_tiling_on_sc=False),
    )(_gather_body)
    return run(idx3, table)


def kernel(indice_sequence, embedding_matrix):
    idx3 = indice_sequence.astype(jnp.int32).reshape(NW, ROWS_PER_W, IDX_MINOR)
    out = _embed(idx3, embedding_matrix)
    return out.reshape(indice_sequence.shape[0], indice_sequence.shape[1], D)
